# Initial kernel scaffold; baseline (speedup 1.0000x reference)
import jax
import jax.numpy as jnp
from jax.experimental import pallas as pl


def kernel(x, edge_index, edge_attr, edge_label_index, node_proj_w, node_proj_b, gcn_w0, gcn_b0, gcn_w1, gcn_b1, enc_w0, enc_b0, enc_w1, enc_b1, dec_w0, dec_b0, dec_w1, dec_b1, dec_w2, dec_b2, ep_w0, ep_b0, ep_w1, ep_b1, ep_w2, ep_b2):
    raise NotImplementedError("write your pallas kernel here")


# speedup vs baseline: 1.9991x; 1.9991x over previous
import functools

import jax
import jax.numpy as jnp
from jax.experimental import pallas as pl


def _Z(*_):
    return jnp.zeros((), jnp.int32)


def _pick_block(m):
    for br in (2000, 1000, 500, 200, 100, 50, 10, 8):
        if m % br == 0:
            return br
    return m


def _mm_body(x_ref, w_ref, b_ref, o_ref, *, relu):
    acc = jnp.dot(x_ref[...], w_ref[...], preferred_element_type=jnp.float32)
    acc = acc + b_ref[...]
    if relu:
        acc = jnp.maximum(acc, 0.0)
    o_ref[...] = acc


def _matmul(x, w, b, relu=False):
    m, k = x.shape
    n = w.shape[1]
    br = _pick_block(m)
    return pl.pallas_call(
        functools.partial(_mm_body, relu=relu),
        grid=(m // br,),
        in_specs=[
            pl.BlockSpec((br, k), lambda i: (i, _Z())),
            pl.BlockSpec((k, n), lambda i: (_Z(), _Z())),
            pl.BlockSpec((1, n), lambda i: (_Z(), _Z())),
        ],
        out_specs=pl.BlockSpec((br, n), lambda i: (i, _Z())),
        out_shape=jax.ShapeDtypeStruct((m, n), jnp.float32),
    )(x, w, b.reshape(1, n))


def _edge_head_body(se_ref, de_ref, ea_ref, val_ref,
                    ew0, eb0, ew1, eb1,
                    dw0, db0, dw1, db1, dw2, db2,
                    pw0, pb0, pw1, pb1, pw2, pb2,
                    lo_ref, fo_ref):
    f32 = jnp.float32
    enc = jnp.maximum(jnp.dot(ea_ref[...], ew0[...], preferred_element_type=f32) + eb0[...], 0.0)
    ee = (jnp.dot(enc, ew1[...], preferred_element_type=f32) + eb1[...]) * val_ref[...]
    ei = jnp.concatenate([se_ref[...], de_ref[...], ee], axis=1)
    z = jnp.maximum(jnp.dot(ei, dw0[...], preferred_element_type=f32) + db0[...], 0.0)
    z = jnp.maximum(jnp.dot(z, dw1[...], preferred_element_type=f32) + db1[...], 0.0)
    lo_ref[...] = jnp.dot(z, dw2[...], preferred_element_type=f32) + db2[...]
    g = jnp.maximum(jnp.dot(ei, pw0[...], preferred_element_type=f32) + pb0[...], 0.0)
    g = jnp.maximum(jnp.dot(g, pw1[...], preferred_element_type=f32) + pb1[...], 0.0)
    fo_ref[...] = jnp.dot(g, pw2[...], preferred_element_type=f32) + pb2[...]


def _edge_head(se, de, ea, val, weights):
    m = se.shape[0]
    br = _pick_block(m)
    grid = (m // br,)

    def row_spec(cols):
        return pl.BlockSpec((br, cols), lambda i: (i, _Z()))

    def w_spec(a):
        r, c = a.shape
        return pl.BlockSpec((r, c), lambda i: (_Z(), _Z()))

    ws = []
    for a in weights:
        a = a.reshape(1, -1) if a.ndim == 1 else a
        ws.append(a)
    return pl.pallas_call(
        _edge_head_body,
        grid=grid,
        in_specs=[row_spec(se.shape[1]), row_spec(de.shape[1]),
                  row_spec(ea.shape[1]), row_spec(1)] + [w_spec(a) for a in ws],
        out_specs=[row_spec(1), row_spec(3)],
        out_shape=[jax.ShapeDtypeStruct((m, 1), jnp.float32),
                   jax.ShapeDtypeStruct((m, 3), jnp.float32)],
    )(se, de, ea, val, *ws)


def kernel(x, edge_index, edge_attr, edge_label_index, node_proj_w, node_proj_b,
           gcn_w0, gcn_b0, gcn_w1, gcn_b1, enc_w0, enc_b0, enc_w1, enc_b1,
           dec_w0, dec_b0, dec_w1, dec_b1, dec_w2, dec_b2,
           ep_w0, ep_b0, ep_w1, ep_b1, ep_w2, ep_b2):
    n = x.shape[0]
    src = edge_index[0]
    dst = edge_index[1]

    h = _matmul(x, node_proj_w, node_proj_b)

    deg = jax.ops.segment_sum(jnp.ones(dst.shape, jnp.float32), dst,
                              num_segments=n) + 1.0
    dinv = jax.lax.rsqrt(deg)

    zero64 = jnp.zeros((64,), jnp.float32)
    for w, b, act in ((gcn_w0, gcn_b0, True), (gcn_w1, gcn_b1, False)):
        hw = _matmul(h, w, zero64)
        g = hw * dinv[:, None]
        s = jax.ops.segment_sum(g[src], dst, num_segments=n)
        h = dinv[:, None] * (s + g) + b
        if act:
            h = jnp.maximum(h, 0.0)

    keys = src * n + dst
    order = jnp.argsort(keys)
    skeys = keys[order]
    qsrc = edge_label_index[0]
    qdst = edge_label_index[1]
    qkeys = qdst * n + qsrc
    pos = jnp.searchsorted(skeys, qkeys, side='right') - 1
    posc = jnp.clip(pos, 0, skeys.shape[0] - 1)
    valid = (pos >= 0) & (skeys[posc] == qkeys)
    eidx = order[posc]

    se = h[qsrc]
    de = h[qdst]
    ea = edge_attr[eidx]
    val = valid.astype(jnp.float32)[:, None]

    weights = (enc_w0, enc_b0, enc_w1, enc_b1,
               dec_w0, dec_b0, dec_w1, dec_b1, dec_w2, dec_b2,
               ep_w0, ep_b0, ep_w1, ep_b1, ep_w2, ep_b2)
    logits, feats = _edge_head(se, de, ea, val, weights)
    return (logits, feats)
